# Initial kernel scaffold; baseline (speedup 1.0000x reference)
import jax
import jax.numpy as jnp
from jax.experimental import pallas as pl


def kernel(proposal_boxes, gt_boxes, gt_classes):
    raise NotImplementedError("write your pallas kernel here")


# speedup vs baseline: 2.5722x; 2.5722x over previous
import jax
import jax.numpy as jnp
from jax import lax
from jax.experimental import pallas as pl
from jax.experimental.pallas import tpu as pltpu
from jax.experimental.pallas import tpu_sc as plsc

_N = 20000
_NPAD = 20480
_M = 128
_NCLS = 80
_NFG = 128
_NBG = 384
_NS = 512
_ROWS = _NPAD // 128
_RB = 32
_GRID = _ROWS // _RB
_NTILES = 16
_CH = _NPAD // _NTILES
_NV = _CH // 16


def _tc_body(p_ref, gt_ref, gtc_ref, vals_ref, idxs_ref, cls_ref):
    px0 = p_ref[0]
    py0 = p_ref[1]
    px1 = p_ref[2]
    py1 = p_ref[3]
    parea = (px1 - px0) * (py1 - py0)
    shape = px0.shape

    def body(j, carry):
        bv, bi, bc = carry
        gx0 = gt_ref[j, 0]
        gy0 = gt_ref[j, 1]
        gx1 = gt_ref[j, 2]
        gy1 = gt_ref[j, 3]
        ga = (gx1 - gx0) * (gy1 - gy0)
        w = jnp.maximum(jnp.minimum(gx1, px1) - jnp.maximum(gx0, px0), 0.0)
        h = jnp.maximum(jnp.minimum(gy1, py1) - jnp.maximum(gy0, py0), 0.0)
        inter = w * h
        union = ga + parea - inter
        iou = jnp.where(union > 0, inter / union, 0.0)
        upd = iou > bv
        bv = jnp.where(upd, iou, bv)
        bi = jnp.where(upd, j, bi)
        bc = jnp.where(upd, gtc_ref[j], bc)
        return bv, bi, bc

    init = (jnp.full(shape, -1.0, jnp.float32),
            jnp.zeros(shape, jnp.int32),
            jnp.zeros(shape, jnp.int32))
    bv, bi, bc = lax.fori_loop(0, _M, body, init)
    vals_ref[...] = bv
    idxs_ref[...] = bi
    cls_ref[...] = jnp.where(bv >= 0.5, bc, _NCLS)


def _match_tc(pc, gt, gtc):
    return pl.pallas_call(
        _tc_body,
        grid=(_GRID,),
        in_specs=[
            pl.BlockSpec((4, _RB, 128), lambda i: (0, i, 0)),
            pl.BlockSpec(memory_space=pltpu.SMEM),
            pl.BlockSpec(memory_space=pltpu.SMEM),
        ],
        out_specs=[
            pl.BlockSpec((_RB, 128), lambda i: (i, 0)),
            pl.BlockSpec((_RB, 128), lambda i: (i, 0)),
            pl.BlockSpec((_RB, 128), lambda i: (i, 0)),
        ],
        out_shape=[
            jax.ShapeDtypeStruct((_ROWS, 128), jnp.float32),
            jax.ShapeDtypeStruct((_ROWS, 128), jnp.int32),
            jax.ShapeDtypeStruct((_ROWS, 128), jnp.int32),
        ],
    )(pc, gt, gtc)


def _sc_body(cls_hbm, midx_hbm, prop_hbm, gt_hbm,
             out_idx, out_cls, out_gtb, out_pb, stg_cnt, stg_tab,
             cls_v, cnt_v, counts_v, lout_v, col_v,
             sidx_v, tgt_v, scls_v, pidx_v, gidx_v, pflat_v, gflat_v,
             gsafe_v, sem):
    cid = lax.axis_index("c")
    tid = lax.axis_index("s")
    on0 = cid == 0
    iota = lax.broadcasted_iota(jnp.int32, (16,), 0)
    zero16 = jnp.zeros((16,), jnp.int32)
    base = tid * _CH

    @pl.when(on0)
    def _pass_count():
        pltpu.sync_copy(cls_hbm.at[pl.ds(base, _CH)], cls_v)
        fc = zero16
        bc = zero16
        for v in range(_NV):
            c16 = cls_v[pl.ds(v * 16, 16)]
            g = iota + (base + v * 16)
            valid = g < _N
            fg = jnp.logical_and(c16 != _NCLS, valid)
            bg = jnp.logical_and(c16 == _NCLS, valid)
            fc = fc + plsc.all_reduce_population_count(fg)
            bc = bc + plsc.all_reduce_population_count(bg)
        cnt_v[...] = jnp.where(iota == 0, fc, jnp.where(iota == 1, bc, 0))
        pltpu.sync_copy(cnt_v, stg_cnt.at[tid])

    plsc.subcore_barrier()

    @pl.when(on0)
    def _pass_scatter():
        pltpu.sync_copy(stg_cnt, counts_v)
        cvec_f = plsc.load_gather(counts_v, [iota, zero16])
        cvec_b = plsc.load_gather(counts_v, [iota, zero16 + 1])
        pre = iota < tid
        offF = jnp.sum(jnp.where(pre, cvec_f, 0))
        offB = jnp.sum(jnp.where(pre, cvec_b, 0))
        Ft = jnp.sum(cvec_f)
        Bt = jnp.sum(cvec_b)
        for i in range(528 // 16):
            lout_v[pl.ds(i * 16, 16)] = zero16
        runF = jnp.broadcast_to(offF, (16,))
        runB = jnp.broadcast_to(offB, (16,))
        FtV = jnp.broadcast_to(Ft, (16,))
        BtV = jnp.broadcast_to(Bt, (16,))
        for v in range(_NV):
            c16 = cls_v[pl.ds(v * 16, 16)]
            g = iota + (base + v * 16)
            valid = g < _N
            fg = jnp.logical_and(c16 != _NCLS, valid)
            bg = jnp.logical_and(c16 == _NCLS, valid)
            fgi = fg.astype(jnp.int32)
            bgi = bg.astype(jnp.int32)
            pF = runF + jnp.cumsum(fgi) - fgi
            pB = runB + jnp.cumsum(bgi) - bgi
            sA = pF
            mA = jnp.logical_and(fg, pF < _NFG)
            sB = FtV + pB
            mB = jnp.logical_and(bg, sB < _NFG)
            sC = pB + _NFG
            mC = jnp.logical_and(bg, pB < _NBG)
            sD = BtV + pF + _NFG
            mD = jnp.logical_and(fg, sD < _NS)
            clamp = lambda s, d: jnp.maximum(jnp.minimum(s, d), 0)
            plsc.store_scatter(lout_v, [clamp(sA, 512)], g, mask=mA)
            plsc.store_scatter(lout_v, [clamp(sB, 513)], g, mask=mB)
            plsc.store_scatter(lout_v, [clamp(sC, 514)], g, mask=mC)
            plsc.store_scatter(lout_v, [clamp(sD, 515)], g, mask=mD)
            runF = runF + plsc.all_reduce_population_count(fg)
            runB = runB + plsc.all_reduce_population_count(bg)
        pltpu.sync_copy(lout_v.at[pl.ds(0, _NS)], stg_tab.at[tid])

    plsc.subcore_barrier()

    @pl.when(on0)
    def _merge_gather():
        pltpu.sync_copy(stg_tab, col_v)
        cbase = tid * 32
        a0 = zero16
        a1 = zero16
        for r in range(_NTILES):
            a0 = a0 + col_v[r, pl.ds(cbase, 16)]
            a1 = a1 + col_v[r, pl.ds(cbase + 16, 16)]
        sidx_v[pl.ds(0, 16)] = a0
        sidx_v[pl.ds(16, 16)] = a1
        a0c = jnp.maximum(jnp.minimum(a0, _NPAD - 1), 0)
        a1c = jnp.maximum(jnp.minimum(a1, _NPAD - 1), 0)
        gsafe_v[pl.ds(0, 16)] = a0c
        gsafe_v[pl.ds(16, 16)] = a1c
        pltpu.sync_copy(sidx_v, out_idx.at[pl.ds(cbase, 32)])
        pltpu.async_copy(cls_hbm.at[gsafe_v], scls_v, sem).wait()
        pltpu.sync_copy(scls_v, out_cls.at[pl.ds(cbase, 32)])
        pltpu.async_copy(midx_hbm.at[gsafe_v], tgt_v, sem).wait()
        t0 = jnp.maximum(jnp.minimum(tgt_v[pl.ds(0, 16)], _M - 1), 0)
        t1 = jnp.maximum(jnp.minimum(tgt_v[pl.ds(16, 16)], _M - 1), 0)
        for c in range(4):
            pidx_v[pl.ds(c * 32, 16)] = a0c * 4 + c
            pidx_v[pl.ds(c * 32 + 16, 16)] = a1c * 4 + c
            gidx_v[pl.ds(c * 32, 16)] = t0 * 4 + c
            gidx_v[pl.ds(c * 32 + 16, 16)] = t1 * 4 + c
        pltpu.async_copy(prop_hbm.at[pidx_v], pflat_v, sem).wait()
        pltpu.async_copy(gt_hbm.at[gidx_v], gflat_v, sem).wait()
        for c in range(4):
            pltpu.sync_copy(pflat_v.at[pl.ds(c * 32, 32)],
                            out_pb.at[c, pl.ds(cbase, 32)])
            pltpu.sync_copy(gflat_v.at[pl.ds(c * 32, 32)],
                            out_gtb.at[c, pl.ds(cbase, 32)])


def _sample_sc(cls_f, midx_f, prop_pad, gt):
    mesh = plsc.VectorSubcoreMesh(core_axis_name="c", subcore_axis_name="s")
    kern = pl.kernel(
        _sc_body,
        mesh=mesh,
        compiler_params=pltpu.CompilerParams(needs_layout_passes=False),
        out_type=[
            jax.ShapeDtypeStruct((_NS,), jnp.int32),
            jax.ShapeDtypeStruct((_NS,), jnp.int32),
            jax.ShapeDtypeStruct((4, _NS), jnp.float32),
            jax.ShapeDtypeStruct((4, _NS), jnp.float32),
            jax.ShapeDtypeStruct((_NTILES, 16), jnp.int32),
            jax.ShapeDtypeStruct((_NTILES, _NS), jnp.int32),
        ],
        scratch_types=[
            pltpu.VMEM((_CH,), jnp.int32),
            pltpu.VMEM((16,), jnp.int32),
            pltpu.VMEM((_NTILES, 16), jnp.int32),
            pltpu.VMEM((528,), jnp.int32),
            pltpu.VMEM((_NTILES, _NS), jnp.int32),
            pltpu.VMEM((32,), jnp.int32),
            pltpu.VMEM((32,), jnp.int32),
            pltpu.VMEM((32,), jnp.int32),
            pltpu.VMEM((128,), jnp.int32),
            pltpu.VMEM((128,), jnp.int32),
            pltpu.VMEM((128,), jnp.float32),
            pltpu.VMEM((128,), jnp.float32),
            pltpu.VMEM((32,), jnp.int32),
            pltpu.SemaphoreType.DMA,
        ],
    )
    outs = kern(cls_f, midx_f, prop_pad.reshape(-1), gt.reshape(-1))
    return outs[:4]


def kernel(proposal_boxes, gt_boxes, gt_classes):
    pb = jnp.asarray(proposal_boxes, jnp.float32)
    gt = jnp.asarray(gt_boxes, jnp.float32)
    gtc = jnp.asarray(gt_classes, jnp.int32)
    pb_pad = jnp.concatenate(
        [pb, jnp.zeros((_NPAD - _N, 4), jnp.float32)], axis=0)
    pc = pb_pad.T.reshape(4, _ROWS, 128)
    vals2, idxs2, cls2 = _match_tc(pc, gt, gtc)
    vals = vals2.reshape(_NPAD)[:_N]
    idxs = idxs2.reshape(_NPAD)[:_N]
    s_idx, s_cls, s_gtb_t, s_pb_t = _sample_sc(
        cls2.reshape(_NPAD), idxs2.reshape(_NPAD), pb_pad, gt)
    return (vals, idxs, s_idx, s_cls, s_gtb_t.T, s_pb_t.T)


# speedup vs baseline: 2.6829x; 1.0431x over previous
import jax
import jax.numpy as jnp
from jax import lax
from jax.experimental import pallas as pl
from jax.experimental.pallas import tpu as pltpu
from jax.experimental.pallas import tpu_sc as plsc

_N = 20000
_NPAD = 20480
_M = 128
_NCLS = 80
_NFG = 128
_NBG = 384
_NS = 512
_ROWS = _NPAD // 128
_RB = 32
_GRID = _ROWS // _RB
_NTILES = 16
_CH = _NPAD // _NTILES
_NV = _CH // 16


def _tc_body(p_ref, gt_ref, gtc_ref, vals_ref, idxs_ref, cls_ref):
    px0 = p_ref[0]
    py0 = p_ref[1]
    px1 = p_ref[2]
    py1 = p_ref[3]
    parea = (px1 - px0) * (py1 - py0)
    shape = px0.shape

    def body(j, carry):
        bv, bi, bc = carry
        gx0 = gt_ref[j, 0]
        gy0 = gt_ref[j, 1]
        gx1 = gt_ref[j, 2]
        gy1 = gt_ref[j, 3]
        ga = (gx1 - gx0) * (gy1 - gy0)
        w = jnp.maximum(jnp.minimum(gx1, px1) - jnp.maximum(gx0, px0), 0.0)
        h = jnp.maximum(jnp.minimum(gy1, py1) - jnp.maximum(gy0, py0), 0.0)
        inter = w * h
        union = ga + parea - inter
        iou = jnp.where(union > 0, inter / union, 0.0)
        upd = iou > bv
        bv = jnp.where(upd, iou, bv)
        bi = jnp.where(upd, j, bi)
        bc = jnp.where(upd, gtc_ref[j], bc)
        return bv, bi, bc

    init = (jnp.full(shape, -1.0, jnp.float32),
            jnp.zeros(shape, jnp.int32),
            jnp.zeros(shape, jnp.int32))
    bv, bi, bc = lax.fori_loop(0, _M, body, init)
    vals_ref[...] = bv
    idxs_ref[...] = bi
    cls_ref[...] = jnp.where(bv >= 0.5, bc, _NCLS)


def _match_tc(pc, gt, gtc):
    return pl.pallas_call(
        _tc_body,
        grid=(_GRID,),
        in_specs=[
            pl.BlockSpec((4, _RB, 128), lambda i: (0, i, 0)),
            pl.BlockSpec(memory_space=pltpu.SMEM),
            pl.BlockSpec(memory_space=pltpu.SMEM),
        ],
        out_specs=[
            pl.BlockSpec((_RB, 128), lambda i: (i, 0)),
            pl.BlockSpec((_RB, 128), lambda i: (i, 0)),
            pl.BlockSpec((_RB, 128), lambda i: (i, 0)),
        ],
        out_shape=[
            jax.ShapeDtypeStruct((_ROWS, 128), jnp.float32),
            jax.ShapeDtypeStruct((_ROWS, 128), jnp.int32),
            jax.ShapeDtypeStruct((_ROWS, 128), jnp.int32),
        ],
    )(pc, gt, gtc)


def _sc_body(cls_hbm, midx_hbm, prop_hbm, gt_hbm,
             out_idx, out_cls, out_gtb, out_pb, stg_cnt, stg_tab,
             cls_v, cnt_v, counts_v, lout_v, col_v,
             sidx_v, tgt_v, scls_v, pidx_v, gidx_v, pflat_v, gflat_v,
             gsafe_v, sem):
    cid = lax.axis_index("c")
    tid = lax.axis_index("s")
    on0 = cid == 0
    iota = lax.broadcasted_iota(jnp.int32, (16,), 0)
    zero16 = jnp.zeros((16,), jnp.int32)
    base = tid * _CH

    @pl.when(on0)
    def _pass_count():
        pltpu.sync_copy(cls_hbm.at[pl.ds(base, _CH)], cls_v)

        def cnt_body(v, carry):
            fc, bc = carry
            c16 = cls_v[pl.ds(v * 16, 16)]
            g = iota + (base + v * 16)
            valid = g < _N
            fg = jnp.logical_and(c16 != _NCLS, valid)
            bg = jnp.logical_and(c16 == _NCLS, valid)
            fc = fc + plsc.all_reduce_population_count(fg)
            bc = bc + plsc.all_reduce_population_count(bg)
            return fc, bc

        fc, bc = lax.fori_loop(0, _NV, cnt_body, (zero16, zero16))
        cnt_v[...] = jnp.where(iota == 0, fc, jnp.where(iota == 1, bc, 0))
        pltpu.sync_copy(cnt_v, stg_cnt.at[tid])

    plsc.subcore_barrier()

    @pl.when(on0)
    def _pass_scatter():
        pltpu.sync_copy(stg_cnt, counts_v)
        cvec_f = plsc.load_gather(counts_v, [iota, zero16])
        cvec_b = plsc.load_gather(counts_v, [iota, zero16 + 1])
        pre = iota < tid
        offF = jnp.sum(jnp.where(pre, cvec_f, 0))
        offB = jnp.sum(jnp.where(pre, cvec_b, 0))
        Ft = jnp.sum(cvec_f)
        Bt = jnp.sum(cvec_b)
        for i in range(528 // 16):
            lout_v[pl.ds(i * 16, 16)] = zero16
        runF = jnp.broadcast_to(offF, (16,))
        runB = jnp.broadcast_to(offB, (16,))
        FtV = jnp.broadcast_to(Ft, (16,))
        BtV = jnp.broadcast_to(Bt, (16,))
        def scat_body(v, carry):
            runF, runB = carry
            c16 = cls_v[pl.ds(v * 16, 16)]
            g = iota + (base + v * 16)
            valid = g < _N
            fg = jnp.logical_and(c16 != _NCLS, valid)
            bg = jnp.logical_and(c16 == _NCLS, valid)
            fgi = fg.astype(jnp.int32)
            bgi = bg.astype(jnp.int32)
            pF = runF + jnp.cumsum(fgi) - fgi
            pB = runB + jnp.cumsum(bgi) - bgi
            sA = pF
            mA = jnp.logical_and(fg, pF < _NFG)
            sB = FtV + pB
            mB = jnp.logical_and(bg, sB < _NFG)
            sC = pB + _NFG
            mC = jnp.logical_and(bg, pB < _NBG)
            sD = BtV + pF + _NFG
            mD = jnp.logical_and(fg, sD < _NS)
            clamp = lambda s, d: jnp.maximum(jnp.minimum(s, d), 0)
            plsc.store_scatter(lout_v, [clamp(sA, 512)], g, mask=mA)
            plsc.store_scatter(lout_v, [clamp(sB, 513)], g, mask=mB)
            plsc.store_scatter(lout_v, [clamp(sC, 514)], g, mask=mC)
            plsc.store_scatter(lout_v, [clamp(sD, 515)], g, mask=mD)
            runF = runF + plsc.all_reduce_population_count(fg)
            runB = runB + plsc.all_reduce_population_count(bg)
            return runF, runB

        lax.fori_loop(0, _NV, scat_body, (runF, runB))
        pltpu.sync_copy(lout_v.at[pl.ds(0, _NS)], stg_tab.at[tid])

    plsc.subcore_barrier()

    @pl.when(on0)
    def _merge_gather():
        pltpu.sync_copy(stg_tab, col_v)
        cbase = tid * 32
        a0 = zero16
        a1 = zero16
        for r in range(_NTILES):
            a0 = a0 + col_v[r, pl.ds(cbase, 16)]
            a1 = a1 + col_v[r, pl.ds(cbase + 16, 16)]
        sidx_v[pl.ds(0, 16)] = a0
        sidx_v[pl.ds(16, 16)] = a1
        a0c = jnp.maximum(jnp.minimum(a0, _NPAD - 1), 0)
        a1c = jnp.maximum(jnp.minimum(a1, _NPAD - 1), 0)
        gsafe_v[pl.ds(0, 16)] = a0c
        gsafe_v[pl.ds(16, 16)] = a1c
        pltpu.sync_copy(sidx_v, out_idx.at[pl.ds(cbase, 32)])
        pltpu.async_copy(cls_hbm.at[gsafe_v], scls_v, sem).wait()
        pltpu.sync_copy(scls_v, out_cls.at[pl.ds(cbase, 32)])
        pltpu.async_copy(midx_hbm.at[gsafe_v], tgt_v, sem).wait()
        t0 = jnp.maximum(jnp.minimum(tgt_v[pl.ds(0, 16)], _M - 1), 0)
        t1 = jnp.maximum(jnp.minimum(tgt_v[pl.ds(16, 16)], _M - 1), 0)
        for c in range(4):
            pidx_v[pl.ds(c * 32, 16)] = a0c * 4 + c
            pidx_v[pl.ds(c * 32 + 16, 16)] = a1c * 4 + c
            gidx_v[pl.ds(c * 32, 16)] = t0 * 4 + c
            gidx_v[pl.ds(c * 32 + 16, 16)] = t1 * 4 + c
        pltpu.async_copy(prop_hbm.at[pidx_v], pflat_v, sem).wait()
        pltpu.async_copy(gt_hbm.at[gidx_v], gflat_v, sem).wait()
        for c in range(4):
            pltpu.sync_copy(pflat_v.at[pl.ds(c * 32, 32)],
                            out_pb.at[c, pl.ds(cbase, 32)])
            pltpu.sync_copy(gflat_v.at[pl.ds(c * 32, 32)],
                            out_gtb.at[c, pl.ds(cbase, 32)])


def _sample_sc(cls_f, midx_f, prop_pad, gt):
    mesh = plsc.VectorSubcoreMesh(core_axis_name="c", subcore_axis_name="s")
    kern = pl.kernel(
        _sc_body,
        mesh=mesh,
        compiler_params=pltpu.CompilerParams(needs_layout_passes=False),
        out_type=[
            jax.ShapeDtypeStruct((_NS,), jnp.int32),
            jax.ShapeDtypeStruct((_NS,), jnp.int32),
            jax.ShapeDtypeStruct((4, _NS), jnp.float32),
            jax.ShapeDtypeStruct((4, _NS), jnp.float32),
            jax.ShapeDtypeStruct((_NTILES, 16), jnp.int32),
            jax.ShapeDtypeStruct((_NTILES, _NS), jnp.int32),
        ],
        scratch_types=[
            pltpu.VMEM((_CH,), jnp.int32),
            pltpu.VMEM((16,), jnp.int32),
            pltpu.VMEM((_NTILES, 16), jnp.int32),
            pltpu.VMEM((528,), jnp.int32),
            pltpu.VMEM((_NTILES, _NS), jnp.int32),
            pltpu.VMEM((32,), jnp.int32),
            pltpu.VMEM((32,), jnp.int32),
            pltpu.VMEM((32,), jnp.int32),
            pltpu.VMEM((128,), jnp.int32),
            pltpu.VMEM((128,), jnp.int32),
            pltpu.VMEM((128,), jnp.float32),
            pltpu.VMEM((128,), jnp.float32),
            pltpu.VMEM((32,), jnp.int32),
            pltpu.SemaphoreType.DMA,
        ],
    )
    outs = kern(cls_f, midx_f, prop_pad.reshape(-1), gt.reshape(-1))
    return outs[:4]


def kernel(proposal_boxes, gt_boxes, gt_classes):
    pb = jnp.asarray(proposal_boxes, jnp.float32)
    gt = jnp.asarray(gt_boxes, jnp.float32)
    gtc = jnp.asarray(gt_classes, jnp.int32)
    pb_pad = jnp.concatenate(
        [pb, jnp.zeros((_NPAD - _N, 4), jnp.float32)], axis=0)
    pc = pb_pad.T.reshape(4, _ROWS, 128)
    vals2, idxs2, cls2 = _match_tc(pc, gt, gtc)
    vals = vals2.reshape(_NPAD)[:_N]
    idxs = idxs2.reshape(_NPAD)[:_N]
    s_idx, s_cls, s_gtb_t, s_pb_t = _sample_sc(
        cls2.reshape(_NPAD), idxs2.reshape(_NPAD), pb_pad, gt)
    return (vals, idxs, s_idx, s_cls, s_gtb_t.T, s_pb_t.T)
